# baseline (speedup 1.0000x reference)
import jax
import jax.numpy as jnp
from jax.experimental import pallas as pl
from jax.experimental.pallas import tpu as pltpu

B, N, D = 8, 2048, 256
CHUNK = 512
NCHUNK = N // CHUNK
BIG = 1 << 30
THRESH2 = 1e-4


def _body(kAB_ref, kBt_ref, kBA_ref, kAt_ref, dA_ref, dB_ref, out_ref):
    a = dA_ref[0]
    b = dB_ref[0]
    na = a * (20.0 / jnp.sqrt(jnp.sum(a * a, axis=-1, keepdims=True)))
    nb = b / jnp.sqrt(jnp.sum(b * b, axis=-1, keepdims=True))

    kBt = kBt_ref[0]
    kAt = kAt_ref[0]

    se_c = jnp.zeros((1, N), jnp.float32)
    se_r_chunks = []
    min_B_chunks = []
    jstar_chunks = []
    corrsel_chunks = []
    min_A_chunks = []
    istar_chunks = []
    for ci in range(NCHUNK):
        r0 = ci * CHUNK
        corr = jax.lax.dot_general(
            na[r0:r0 + CHUNK], nb, (((1,), (1,)), ((), ())),
            preferred_element_type=jnp.float32)
        e = jnp.exp(corr)
        se_r_chunks.append(jnp.sum(e, axis=1, keepdims=True))
        se_c = se_c + jnp.sum(e, axis=0, keepdims=True)

        iota_m = jax.lax.broadcasted_iota(jnp.int32, (CHUNK, N), 1)

        dx = kAB_ref[0, r0:r0 + CHUNK, 0:1] - kBt[0:1, :]
        dy = kAB_ref[0, r0:r0 + CHUNK, 1:2] - kBt[1:2, :]
        d2B = dx * dx + dy * dy
        mB = jnp.min(d2B, axis=1, keepdims=True)
        min_B_chunks.append(mB)
        selB = d2B == mB
        jstar_chunks.append(jnp.min(jnp.where(selB, iota_m, BIG), axis=1,
                                    keepdims=True))
        corrsel_chunks.append(jnp.max(
            jnp.where(selB, corr, -jnp.inf), axis=1, keepdims=True))

        dx = kBA_ref[0, r0:r0 + CHUNK, 0:1] - kAt[0:1, :]
        dy = kBA_ref[0, r0:r0 + CHUNK, 1:2] - kAt[1:2, :]
        d2A = dx * dx + dy * dy
        mA = jnp.min(d2A, axis=1, keepdims=True)
        min_A_chunks.append(mA)
        istar_chunks.append(jnp.min(jnp.where(d2A == mA, iota_m, BIG),
                                    axis=1, keepdims=True))

    lse_c = jnp.log(se_c)
    istar = jnp.concatenate(istar_chunks, axis=0)
    istar_hi = jax.lax.shift_right_logical(istar, 6).astype(jnp.float32)
    istar_lo = jnp.bitwise_and(istar, 63).astype(jnp.float32)
    min_A = jnp.concatenate(min_A_chunks, axis=0)
    stacked = jnp.concatenate(
        [istar_hi, istar_lo, min_A, jnp.zeros((N, 5), jnp.float32)],
        axis=1)

    num = jnp.float32(0.0)
    cnt = jnp.float32(0.0)
    matched = jnp.zeros((1, N), jnp.float32)
    for ci in range(NCHUNK):
        r0 = ci * CHUNK
        iota_m = jax.lax.broadcasted_iota(jnp.int32, (CHUNK, N), 1)
        onehot = (iota_m == jstar_chunks[ci]).astype(jnp.float32)
        g = jnp.dot(onehot, stacked, preferred_element_type=jnp.float32)
        rowf = (jax.lax.broadcasted_iota(jnp.int32, (CHUNK, 1), 0)
                + r0).astype(jnp.float32)
        gi = g[:, 0:1] * 64.0 + g[:, 1:2]
        mutual = ((jnp.abs(gi - rowf) < 0.5)
                  & (min_B_chunks[ci] < THRESH2)
                  & (g[:, 2:3] < THRESH2))
        mutf = mutual.astype(jnp.float32)
        lse_r = jnp.log(se_r_chunks[ci])
        num = num + jnp.sum(mutf * (2.0 * corrsel_chunks[ci] - lse_r))
        cnt = cnt + jnp.sum(mutf)
        matched = matched + jax.lax.dot_general(
            mutf, onehot, (((0,), (0,)), ((), ())),
            preferred_element_type=jnp.float32)
    matched = jnp.where(matched > 0.5, 1.0, 0.0)
    num = num - jnp.sum(matched * lse_c)

    lane = jax.lax.broadcasted_iota(jnp.int32, (1, 1, 128), 2)
    out_ref[...] = jnp.where(lane == 0, num, cnt)


@jax.jit
def kernel(kpts_A, kpts_B, kpts_A_to_B, kpts_B_to_A,
           descriptions_A, descriptions_B):
    kBt = jnp.swapaxes(kpts_B, 1, 2)
    kAt = jnp.swapaxes(kpts_A, 1, 2)

    batch_spec = lambda shp: pl.BlockSpec((1,) + shp, lambda i: (i, 0, 0))
    out = pl.pallas_call(
        _body,
        grid=(B,),
        in_specs=[
            batch_spec((N, 2)),
            batch_spec((2, N)),
            batch_spec((N, 2)),
            batch_spec((2, N)),
            batch_spec((N, D)),
            batch_spec((N, D)),
        ],
        out_specs=pl.BlockSpec((1, 1, 128), lambda i: (i, 0, 0)),
        out_shape=jax.ShapeDtypeStruct((B, 1, 128), jnp.float32),
    )(kpts_A_to_B, kBt, kpts_B_to_A, kAt, descriptions_A, descriptions_B)

    total_num = jnp.sum(out[:, 0, 0])
    total_cnt = jnp.sum(out[:, 0, 1])
    return -total_num / jnp.maximum(total_cnt, 1.0)


# speedup vs baseline: 1.1981x; 1.1981x over previous
import functools

import jax
import jax.numpy as jnp
from jax import lax
from jax.experimental import pallas as pl
from jax.experimental.pallas import tpu as pltpu
from jax.experimental.pallas import tpu_sc as plsc

B, N, D = 8, 2048, 256
CHUNK = 512
NCHUNK = N // CHUNK
BIG = 1 << 30
THRESH2 = 1e-4

NUM_TILES = 32
TILES_PER_BATCH = NUM_TILES // B
ROWS_PER_TILE = N // TILES_PER_BATCH
LANES = 16
VECS_PER_TILE = ROWS_PER_TILE // LANES


def _tc_body(kAB_ref, kBt_ref, kBA_ref, kAt_ref, dA_ref, dB_ref,
             js_ref, cs_ref, lr_ref, mb_ref, is_ref, ma_ref, lc_ref):
    a = dA_ref[0]
    b = dB_ref[0]
    na = a * (20.0 / jnp.sqrt(jnp.sum(a * a, axis=-1, keepdims=True)))
    nb = b / jnp.sqrt(jnp.sum(b * b, axis=-1, keepdims=True))

    kBt = kBt_ref[0]
    kAt = kAt_ref[0]

    se_c = jnp.zeros((1, N), jnp.float32)
    se_r_chunks = []
    min_B_chunks = []
    jstar_chunks = []
    corrsel_chunks = []
    min_A_chunks = []
    istar_chunks = []
    for ci in range(NCHUNK):
        r0 = ci * CHUNK
        corr = jax.lax.dot_general(
            na[r0:r0 + CHUNK], nb, (((1,), (1,)), ((), ())),
            preferred_element_type=jnp.float32)
        e = jnp.exp(corr)
        se_r_chunks.append(jnp.sum(e, axis=1, keepdims=True))
        se_c = se_c + jnp.sum(e, axis=0, keepdims=True)

        iota_m = jax.lax.broadcasted_iota(jnp.int32, (CHUNK, N), 1)

        dx = kAB_ref[0, r0:r0 + CHUNK, 0:1] - kBt[0:1, :]
        dy = kAB_ref[0, r0:r0 + CHUNK, 1:2] - kBt[1:2, :]
        d2B = dx * dx + dy * dy
        mB = jnp.min(d2B, axis=1, keepdims=True)
        min_B_chunks.append(mB)
        selB = d2B == mB
        jstar_chunks.append(jnp.min(jnp.where(selB, iota_m, BIG), axis=1,
                                    keepdims=True))
        corrsel_chunks.append(jnp.max(
            jnp.where(selB, corr, -jnp.inf), axis=1, keepdims=True))

        dx = kBA_ref[0, r0:r0 + CHUNK, 0:1] - kAt[0:1, :]
        dy = kBA_ref[0, r0:r0 + CHUNK, 1:2] - kAt[1:2, :]
        d2A = dx * dx + dy * dy
        mA = jnp.min(d2A, axis=1, keepdims=True)
        min_A_chunks.append(mA)
        istar_chunks.append(jnp.min(jnp.where(d2A == mA, iota_m, BIG),
                                    axis=1, keepdims=True))

    js_ref[...] = jnp.concatenate(jstar_chunks, axis=0).astype(
        jnp.float32).reshape(1, N, 1)
    cs_ref[...] = jnp.concatenate(corrsel_chunks, axis=0).reshape(1, N, 1)
    lr_ref[...] = jnp.log(jnp.concatenate(se_r_chunks, axis=0)).reshape(
        1, N, 1)
    mb_ref[...] = jnp.concatenate(min_B_chunks, axis=0).reshape(1, N, 1)
    is_ref[...] = jnp.concatenate(istar_chunks, axis=0).astype(
        jnp.float32).reshape(1, N, 1)
    ma_ref[...] = jnp.concatenate(min_A_chunks, axis=0).reshape(1, N, 1)
    lc_ref[...] = jnp.log(se_c).reshape(1, 1, N)


def _sc_body(js_h, cs_h, lr_h, mb_h, is_h, ma_h, lc_h, out_h,
             istar_v, minA_v, lsec_v, js_v, cs_v, lr_v, mb_v, out_v):
    info = plsc.get_sparse_core_info()
    wid = lax.axis_index("s") * info.num_cores + lax.axis_index("c")
    batch = wid // TILES_PER_BATCH
    base = (wid % TILES_PER_BATCH) * ROWS_PER_TILE

    pltpu.sync_copy(is_h.at[batch], istar_v)
    pltpu.sync_copy(ma_h.at[batch], minA_v)
    pltpu.sync_copy(lc_h.at[batch], lsec_v)
    pltpu.sync_copy(js_h.at[batch, pl.ds(base, ROWS_PER_TILE)], js_v)
    pltpu.sync_copy(cs_h.at[batch, pl.ds(base, ROWS_PER_TILE)], cs_v)
    pltpu.sync_copy(lr_h.at[batch, pl.ds(base, ROWS_PER_TILE)], lr_v)
    pltpu.sync_copy(mb_h.at[batch, pl.ds(base, ROWS_PER_TILE)], mb_v)

    accn = jnp.zeros((LANES,), jnp.float32)
    accc = jnp.zeros((LANES,), jnp.float32)
    iota = lax.broadcasted_iota(jnp.int32, (LANES,), 0)
    for k in range(VECS_PER_TILE):
        o = k * LANES
        idx = js_v[pl.ds(o, LANES)].astype(jnp.int32)
        gi = plsc.load_gather(istar_v, [idx])
        gma = plsc.load_gather(minA_v, [idx])
        glc = plsc.load_gather(lsec_v, [idx])
        rowi = iota + (base + o)
        mut = ((gi.astype(jnp.int32) == rowi)
               & (mb_v[pl.ds(o, LANES)] < THRESH2)
               & (gma < THRESH2))
        val = 2.0 * cs_v[pl.ds(o, LANES)] - lr_v[pl.ds(o, LANES)] - glc
        accn = accn + jnp.where(mut, val, 0.0)
        accc = accc + jnp.where(mut, 1.0, 0.0)
    out_v[pl.ds(0, LANES)] = accn
    out_v[pl.ds(LANES, LANES)] = accc
    pltpu.sync_copy(out_v, out_h.at[wid])


@functools.partial(
    pl.kernel,
    out_type=jax.ShapeDtypeStruct((NUM_TILES, 2 * LANES), jnp.float32),
    mesh=plsc.VectorSubcoreMesh(core_axis_name="c", subcore_axis_name="s"),
    compiler_params=pltpu.CompilerParams(needs_layout_passes=False),
    scratch_types=[
        pltpu.VMEM((N,), jnp.float32),
        pltpu.VMEM((N,), jnp.float32),
        pltpu.VMEM((N,), jnp.float32),
        pltpu.VMEM((ROWS_PER_TILE,), jnp.float32),
        pltpu.VMEM((ROWS_PER_TILE,), jnp.float32),
        pltpu.VMEM((ROWS_PER_TILE,), jnp.float32),
        pltpu.VMEM((ROWS_PER_TILE,), jnp.float32),
        pltpu.VMEM((2 * LANES,), jnp.float32),
    ],
)
def _sc_epilogue(js_h, cs_h, lr_h, mb_h, is_h, ma_h, lc_h, out_h, *scratch):
    _sc_body(js_h, cs_h, lr_h, mb_h, is_h, ma_h, lc_h, out_h, *scratch)


@jax.jit
def kernel(kpts_A, kpts_B, kpts_A_to_B, kpts_B_to_A,
           descriptions_A, descriptions_B):
    kBt = jnp.swapaxes(kpts_B, 1, 2)
    kAt = jnp.swapaxes(kpts_A, 1, 2)

    batch_spec = lambda shp: pl.BlockSpec((1,) + shp, lambda i: (i, 0, 0))
    vec_out = [jax.ShapeDtypeStruct((B, N, 1), jnp.float32)] * 6 + [
        jax.ShapeDtypeStruct((B, 1, N), jnp.float32)]
    vec_spec = [batch_spec((N, 1))] * 6 + [batch_spec((1, N))]
    js, cs, lr, mb, is_, ma, lc = pl.pallas_call(
        _tc_body,
        grid=(B,),
        in_specs=[
            batch_spec((N, 2)),
            batch_spec((2, N)),
            batch_spec((N, 2)),
            batch_spec((2, N)),
            batch_spec((N, D)),
            batch_spec((N, D)),
        ],
        out_specs=vec_spec,
        out_shape=vec_out,
    )(kpts_A_to_B, kBt, kpts_B_to_A, kAt, descriptions_A, descriptions_B)

    out = _sc_epilogue(js.reshape(B, N), cs.reshape(B, N),
                       lr.reshape(B, N), mb.reshape(B, N),
                       is_.reshape(B, N), ma.reshape(B, N),
                       lc.reshape(B, N))
    total_num = jnp.sum(out[:, :LANES])
    total_cnt = jnp.sum(out[:, LANES:])
    return -total_num / jnp.maximum(total_cnt, 1.0)
